# baseline (speedup 1.0000x reference)
import functools

import jax
import jax.numpy as jnp
from jax import lax
from jax.experimental import pallas as pl
from jax.experimental.pallas import tpu as pltpu
from jax.experimental.pallas import tpu_sc as plsc


def _project_table(emb, W, b2):
    n, hid = emb.shape
    blk = 8192

    def body(e_ref, w_ref, b_ref, o0_ref, o1_ref):
        e = e_ref[...]
        o0_ref[...] = jnp.sum(e * w_ref[0:1, :], axis=1) + b_ref[0, 0]
        o1_ref[...] = jnp.sum(e * w_ref[1:2, :], axis=1) + b_ref[0, 1]

    return pl.pallas_call(
        body,
        grid=(pl.cdiv(n, blk),),
        in_specs=[
            pl.BlockSpec((blk, hid), lambda i: (i, 0)),
            pl.BlockSpec((2, hid), lambda i: (0, 0)),
            pl.BlockSpec((1, 2), lambda i: (0, 0)),
        ],
        out_specs=[
            pl.BlockSpec((blk,), lambda i: (i,)),
            pl.BlockSpec((blk,), lambda i: (i,)),
        ],
        out_shape=[
            jax.ShapeDtypeStruct((n,), jnp.float32),
            jax.ShapeDtypeStruct((n,), jnp.float32),
        ],
    )(emb, W, b2)


def _sc_gather(t0, t1, hseg, nc, ns, nch):
    nw = nc * ns
    mesh = plsc.VectorSubcoreMesh(core_axis_name="c", subcore_axis_name="s")

    @functools.partial(
        pl.kernel,
        mesh=mesh,
        out_type=(
            jax.ShapeDtypeStruct((nw, nch, 16, 128), jnp.float32),
            jax.ShapeDtypeStruct((nw, nch, 16, 128), jnp.float32),
        ),
        scratch_types=[
            pltpu.VMEM((16, 128), jnp.int32),
            pltpu.VMEM((16, 128), jnp.float32),
            pltpu.VMEM((16, 128), jnp.float32),
            pltpu.SemaphoreType.DMA,
        ],
        compiler_params=pltpu.CompilerParams(use_tc_tiling_on_sc=False),
    )
    def k(t0_hbm, t1_hbm, h_hbm, o0_hbm, o1_hbm, idx_v, g0_v, g1_v, sem):
        wid = lax.axis_index("s") * nc + lax.axis_index("c")

        def chunk(c, carry):
            pltpu.sync_copy(h_hbm.at[wid, c], idx_v)
            cps = []
            for j in range(16):
                cps.append(pltpu.make_async_copy(
                    t0_hbm.at[idx_v.at[j]], g0_v.at[j], sem))
                cps.append(pltpu.make_async_copy(
                    t1_hbm.at[idx_v.at[j]], g1_v.at[j], sem))
            for cp in cps:
                cp.start()
            for cp in cps:
                cp.wait()
            pltpu.sync_copy(g0_v, o0_hbm.at[wid, c])
            pltpu.sync_copy(g1_v, o1_hbm.at[wid, c])
            return carry

        lax.fori_loop(0, nch, chunk, 0)

    return k(t0, t1, hseg)


def kernel(h, emb, W, b):
    B, L = h.shape
    t0, t1 = _project_table(emb, W, b.reshape(1, -1))
    nc, ns = 2, 16
    nw = nc * ns
    total = B * L
    nch = total // (nw * 16 * 128)
    hseg = h.reshape(nw, nch, 16, 128).astype(jnp.int32)
    o0, o1 = _sc_gather(t0, t1, hseg, nc, ns, nch)
    return jnp.stack(
        [o0.reshape(B, L), o1.reshape(B, L)], axis=-1)


# speedup vs baseline: 1.7250x; 1.7250x over previous
import functools

import jax
import jax.numpy as jnp
from jax import lax
from jax.experimental import pallas as pl
from jax.experimental.pallas import tpu as pltpu
from jax.experimental.pallas import tpu_sc as plsc


def _project_table(emb, W, b2):
    n, hid = emb.shape
    blk = 8192

    def body(e_ref, w_ref, b_ref, o0_ref, o1_ref):
        c = jax.lax.dot_general(
            w_ref[...], e_ref[...], (((1,), (1,)), ((), ())),
            preferred_element_type=jnp.float32,
        )
        o0_ref[...] = c[0] + b_ref[0, 0]
        o1_ref[...] = c[1] + b_ref[0, 1]

    return pl.pallas_call(
        body,
        grid=(pl.cdiv(n, blk),),
        in_specs=[
            pl.BlockSpec((blk, hid), lambda i: (i, 0)),
            pl.BlockSpec((2, hid), lambda i: (0, 0)),
            pl.BlockSpec((1, 2), lambda i: (0, 0)),
        ],
        out_specs=[
            pl.BlockSpec((blk,), lambda i: (i,)),
            pl.BlockSpec((blk,), lambda i: (i,)),
        ],
        out_shape=[
            jax.ShapeDtypeStruct((n,), jnp.float32),
            jax.ShapeDtypeStruct((n,), jnp.float32),
        ],
    )(emb, W, b2)


def _sc_gather(t0, t1, hseg, nc, ns, nch):
    nw = nc * ns
    mesh = plsc.VectorSubcoreMesh(core_axis_name="c", subcore_axis_name="s")

    @functools.partial(
        pl.kernel,
        mesh=mesh,
        out_type=(
            jax.ShapeDtypeStruct((nw, nch, 16, 128), jnp.float32),
            jax.ShapeDtypeStruct((nw, nch, 16, 128), jnp.float32),
        ),
        scratch_types=[
            pltpu.VMEM((16, 128), jnp.int32),
            pltpu.VMEM((16, 128), jnp.float32),
            pltpu.VMEM((16, 128), jnp.float32),
            pltpu.SemaphoreType.DMA,
        ],
        compiler_params=pltpu.CompilerParams(use_tc_tiling_on_sc=False),
    )
    def k(t0_hbm, t1_hbm, h_hbm, o0_hbm, o1_hbm, idx_v, g0_v, g1_v, sem):
        wid = lax.axis_index("s") * nc + lax.axis_index("c")

        def chunk(c, carry):
            pltpu.sync_copy(h_hbm.at[wid, c], idx_v)
            cps = []
            for j in range(16):
                cps.append(pltpu.make_async_copy(
                    t0_hbm.at[idx_v.at[j]], g0_v.at[j], sem))
                cps.append(pltpu.make_async_copy(
                    t1_hbm.at[idx_v.at[j]], g1_v.at[j], sem))
            for cp in cps:
                cp.start()
            for cp in cps:
                cp.wait()
            pltpu.sync_copy(g0_v, o0_hbm.at[wid, c])
            pltpu.sync_copy(g1_v, o1_hbm.at[wid, c])
            return carry

        lax.fori_loop(0, nch, chunk, 0)

    return k(t0, t1, hseg)


def kernel(h, emb, W, b):
    B, L = h.shape
    t0, t1 = _project_table(emb, W, b.reshape(1, -1))
    nc, ns = 2, 16
    nw = nc * ns
    total = B * L
    nch = total // (nw * 16 * 128)
    hseg = h.reshape(nw, nch, 16, 128).astype(jnp.int32)
    o0, o1 = _sc_gather(t0, t1, hseg, nc, ns, nch)
    return jnp.stack(
        [o0.reshape(B, L), o1.reshape(B, L)], axis=-1)


# speedup vs baseline: 1.7323x; 1.0042x over previous
import functools

import jax
import jax.numpy as jnp
from jax import lax
from jax.experimental import pallas as pl
from jax.experimental.pallas import tpu as pltpu
from jax.experimental.pallas import tpu_sc as plsc


def _project_table(emb, W, b2):
    n, hid = emb.shape
    blk = 8192

    def body(e_ref, w_ref, b_ref, o0_ref, o1_ref):
        c = jax.lax.dot_general(
            w_ref[...], e_ref[...], (((1,), (1,)), ((), ())),
            preferred_element_type=jnp.float32,
        )
        o0_ref[...] = c[0] + b_ref[0, 0]
        o1_ref[...] = c[1] + b_ref[0, 1]

    return pl.pallas_call(
        body,
        grid=(pl.cdiv(n, blk),),
        in_specs=[
            pl.BlockSpec((blk, hid), lambda i: (i, 0)),
            pl.BlockSpec((2, hid), lambda i: (0, 0)),
            pl.BlockSpec((1, 2), lambda i: (0, 0)),
        ],
        out_specs=[
            pl.BlockSpec((blk,), lambda i: (i,)),
            pl.BlockSpec((blk,), lambda i: (i,)),
        ],
        out_shape=[
            jax.ShapeDtypeStruct((n,), jnp.float32),
            jax.ShapeDtypeStruct((n,), jnp.float32),
        ],
    )(emb, W, b2)


def _sc_gather(t0, t1, hseg, nc, ns, nch, chunk_len):
    nw = nc * ns
    per_w = nch * chunk_len
    mesh = plsc.VectorSubcoreMesh(core_axis_name="c", subcore_axis_name="s")

    @functools.partial(
        pl.kernel,
        mesh=mesh,
        out_type=(
            jax.ShapeDtypeStruct((nw, per_w), jnp.float32),
            jax.ShapeDtypeStruct((nw, per_w), jnp.float32),
        ),
        scratch_types=[
            pltpu.VMEM((chunk_len,), jnp.int32),
            pltpu.VMEM((chunk_len,), jnp.float32),
            pltpu.VMEM((chunk_len,), jnp.float32),
            pltpu.SemaphoreType.DMA,
        ],
        compiler_params=pltpu.CompilerParams(use_tc_tiling_on_sc=False),
    )
    def k(t0_hbm, t1_hbm, h_hbm, o0_hbm, o1_hbm, idx_v, g0_v, g1_v, sem):
        wid = lax.axis_index("s") * nc + lax.axis_index("c")

        def chunk(c, carry):
            off = c * chunk_len
            pltpu.sync_copy(h_hbm.at[wid, pl.ds(off, chunk_len)], idx_v)
            cps = [
                pltpu.make_async_copy(t0_hbm.at[idx_v], g0_v, sem),
                pltpu.make_async_copy(t1_hbm.at[idx_v], g1_v, sem),
            ]
            for cp in cps:
                cp.start()
            for cp in cps:
                cp.wait()
            pltpu.sync_copy(g0_v, o0_hbm.at[wid, pl.ds(off, chunk_len)])
            pltpu.sync_copy(g1_v, o1_hbm.at[wid, pl.ds(off, chunk_len)])
            return carry

        lax.fori_loop(0, nch, chunk, 0)

    return k(t0, t1, hseg)


def kernel(h, emb, W, b):
    B, L = h.shape
    t0, t1 = _project_table(emb, W, b.reshape(1, -1))
    nc, ns = 2, 16
    nw = nc * ns
    total = B * L
    chunk_len = 2048
    nch = total // (nw * chunk_len)
    hseg = h.reshape(nw, nch * chunk_len).astype(jnp.int32)
    o0, o1 = _sc_gather(t0, t1, hseg, nc, ns, nch, chunk_len)
    return jnp.stack(
        [o0.reshape(B, L), o1.reshape(B, L)], axis=-1)


# speedup vs baseline: 1.9284x; 1.1132x over previous
import functools

import jax
import jax.numpy as jnp
from jax import lax
from jax.experimental import pallas as pl
from jax.experimental.pallas import tpu as pltpu
from jax.experimental.pallas import tpu_sc as plsc


def _project_table(emb, W, b2):
    n, hid = emb.shape
    blk = 8192

    def body(e_ref, w_ref, b_ref, o_ref):
        c = jax.lax.dot_general(
            w_ref[...], e_ref[...], (((1,), (1,)), ((), ())),
            preferred_element_type=jnp.float32,
        )
        r0 = jax.lax.bitcast_convert_type(c[0] + b_ref[0, 0], jnp.int32)
        r1 = jax.lax.bitcast_convert_type(c[1] + b_ref[0, 1], jnp.int32)
        h0 = (r0 + 0x7FFF + ((r0 >> 16) & 1)) >> 16
        h1 = (r1 + 0x7FFF + ((r1 >> 16) & 1)) >> 16
        o_ref[...] = (h1 << 16) | (h0 & 0xFFFF)

    return pl.pallas_call(
        body,
        grid=(pl.cdiv(n, blk),),
        in_specs=[
            pl.BlockSpec((blk, hid), lambda i: (i, 0)),
            pl.BlockSpec((2, hid), lambda i: (0, 0)),
            pl.BlockSpec((1, 2), lambda i: (0, 0)),
        ],
        out_specs=pl.BlockSpec((blk,), lambda i: (i,)),
        out_shape=jax.ShapeDtypeStruct((n,), jnp.int32),
    )(emb, W, b2)


def _sc_gather(t01, hseg, nc, ns, nch, chunk_len):
    nw = nc * ns
    per_w = nch * chunk_len
    mesh = plsc.VectorSubcoreMesh(core_axis_name="c", subcore_axis_name="s")

    @functools.partial(
        pl.kernel,
        mesh=mesh,
        out_type=jax.ShapeDtypeStruct((nw, per_w), jnp.int32),
        scratch_types=[
            pltpu.VMEM((chunk_len,), jnp.int32),
            pltpu.VMEM((chunk_len,), jnp.int32),
            pltpu.SemaphoreType.DMA,
        ],
        compiler_params=pltpu.CompilerParams(use_tc_tiling_on_sc=False),
    )
    def k(t_hbm, h_hbm, out_hbm, idx_v, g_v, sem):
        wid = lax.axis_index("s") * nc + lax.axis_index("c")

        def chunk(c, carry):
            off = c * chunk_len
            pltpu.sync_copy(h_hbm.at[wid, pl.ds(off, chunk_len)], idx_v)
            pltpu.async_copy(t_hbm.at[idx_v], g_v, sem).wait()
            pltpu.sync_copy(g_v, out_hbm.at[wid, pl.ds(off, chunk_len)])
            return carry

        lax.fori_loop(0, nch, chunk, 0)

    return k(t01, hseg)


def kernel(h, emb, W, b):
    B, L = h.shape
    t01 = _project_table(emb, W, b.reshape(1, -1))
    nc, ns = 2, 16
    nw = nc * ns
    total = B * L
    chunk_len = 2048
    nch = total // (nw * chunk_len)
    hseg = h.reshape(nw, nch * chunk_len).astype(jnp.int32)
    g = _sc_gather(t01, hseg, nc, ns, nch, chunk_len).reshape(B, L)
    f0 = jax.lax.bitcast_convert_type(g << 16, jnp.float32)
    f1 = jax.lax.bitcast_convert_type(g & jnp.int32(-65536), jnp.float32)
    return jnp.stack([f0, f1], axis=-1)


# speedup vs baseline: 3.7901x; 1.9654x over previous
import functools

import jax
import jax.numpy as jnp
from jax import lax
from jax.experimental import pallas as pl
from jax.experimental.pallas import tpu as pltpu
from jax.experimental.pallas import tpu_sc as plsc


def _project_table(embT, W, b2):
    hid, n = embT.shape
    blk = 8192

    def body(e_ref, w_ref, b_ref, o_ref):
        c = jax.lax.dot_general(
            w_ref[...], e_ref[...], (((1,), (0,)), ((), ())),
            preferred_element_type=jnp.float32,
        )
        r0 = jax.lax.bitcast_convert_type(c[0] + b_ref[0, 0], jnp.int32)
        r1 = jax.lax.bitcast_convert_type(c[1] + b_ref[0, 1], jnp.int32)
        h0 = (r0 + 0x7FFF + ((r0 >> 16) & 1)) >> 16
        h1 = (r1 + 0x7FFF + ((r1 >> 16) & 1)) >> 16
        o_ref[...] = (h1 << 16) | (h0 & 0xFFFF)

    return pl.pallas_call(
        body,
        grid=(pl.cdiv(n, blk),),
        in_specs=[
            pl.BlockSpec((hid, blk), lambda i: (0, i)),
            pl.BlockSpec((2, hid), lambda i: (0, 0)),
            pl.BlockSpec((1, 2), lambda i: (0, 0)),
        ],
        out_specs=pl.BlockSpec((blk,), lambda i: (i,)),
        out_shape=jax.ShapeDtypeStruct((n,), jnp.int32),
    )(embT, W, b2)


def _sc_gather(t01, hseg, nc, ns, nch, chunk_len):
    nw = nc * ns
    per_w = nch * chunk_len
    mesh = plsc.VectorSubcoreMesh(core_axis_name="c", subcore_axis_name="s")

    @functools.partial(
        pl.kernel,
        mesh=mesh,
        out_type=jax.ShapeDtypeStruct((nw, per_w), jnp.int32),
        scratch_types=[
            pltpu.VMEM((chunk_len,), jnp.int32),
            pltpu.VMEM((chunk_len,), jnp.int32),
            pltpu.SemaphoreType.DMA,
        ],
        compiler_params=pltpu.CompilerParams(use_tc_tiling_on_sc=False),
    )
    def k(t_hbm, h_hbm, out_hbm, idx_v, g_v, sem):
        wid = lax.axis_index("s") * nc + lax.axis_index("c")

        def chunk(c, carry):
            off = c * chunk_len
            pltpu.sync_copy(h_hbm.at[wid, pl.ds(off, chunk_len)], idx_v)
            pltpu.async_copy(t_hbm.at[idx_v], g_v, sem).wait()
            pltpu.sync_copy(g_v, out_hbm.at[wid, pl.ds(off, chunk_len)])
            return carry

        lax.fori_loop(0, nch, chunk, 0)

    return k(t01, hseg)


def kernel(h, emb, W, b):
    B, L = h.shape
    t01 = _project_table(emb.T, W, b.reshape(1, -1))
    nc, ns = 2, 16
    nw = nc * ns
    total = B * L
    chunk_len = 2048
    nch = total // (nw * chunk_len)
    hseg = h.T.astype(jnp.int32).reshape(nw, nch * chunk_len)
    g = _sc_gather(t01, hseg, nc, ns, nch, chunk_len).reshape(L, B)
    f0 = jax.lax.bitcast_convert_type(g << 16, jnp.float32)
    f1 = jax.lax.bitcast_convert_type(g & jnp.int32(-65536), jnp.float32)
    return jnp.stack([f0, f1], axis=1).transpose(2, 0, 1)


# speedup vs baseline: 4.0073x; 1.0573x over previous
import functools

import jax
import jax.numpy as jnp
from jax import lax
from jax.experimental import pallas as pl
from jax.experimental.pallas import tpu as pltpu
from jax.experimental.pallas import tpu_sc as plsc


def _project_table(embT, W, b2):
    hid, n = embT.shape
    blk = 8192

    def body(e_ref, w_ref, b_ref, o_ref):
        c = jax.lax.dot_general(
            w_ref[...], e_ref[...], (((1,), (0,)), ((), ())),
            preferred_element_type=jnp.float32,
        )
        r0 = jax.lax.bitcast_convert_type(c[0] + b_ref[0, 0], jnp.int32)
        r1 = jax.lax.bitcast_convert_type(c[1] + b_ref[0, 1], jnp.int32)
        h0 = (r0 + 0x7FFF + ((r0 >> 16) & 1)) >> 16
        h1 = (r1 + 0x7FFF + ((r1 >> 16) & 1)) >> 16
        o_ref[...] = (h1 << 16) | (h0 & 0xFFFF)

    return pl.pallas_call(
        body,
        grid=(pl.cdiv(n, blk),),
        in_specs=[
            pl.BlockSpec((hid, blk), lambda i: (0, i)),
            pl.BlockSpec((2, hid), lambda i: (0, 0)),
            pl.BlockSpec((1, 2), lambda i: (0, 0)),
        ],
        out_specs=pl.BlockSpec((blk,), lambda i: (i,)),
        out_shape=jax.ShapeDtypeStruct((n,), jnp.int32),
    )(embT, W, b2)


def _sc_gather(t01, hseg, nc, ns, nch, chunk_len):
    nw = nc * ns
    per_w = nch * chunk_len
    mesh = plsc.VectorSubcoreMesh(core_axis_name="c", subcore_axis_name="s")

    @functools.partial(
        pl.kernel,
        mesh=mesh,
        out_type=jax.ShapeDtypeStruct((nw * per_w,), jnp.int32),
        scratch_types=[
            pltpu.VMEM((chunk_len,), jnp.int32),
            pltpu.VMEM((chunk_len,), jnp.int32),
            pltpu.SemaphoreType.DMA,
        ],
        compiler_params=pltpu.CompilerParams(use_tc_tiling_on_sc=False),
    )
    def k(t_hbm, h_hbm, out_hbm, idx_v, g_v, sem):
        wid = lax.axis_index("s") * nc + lax.axis_index("c")
        base = wid * per_w

        def chunk(c, carry):
            off = base + c * chunk_len
            pltpu.sync_copy(h_hbm.at[pl.ds(off, chunk_len)], idx_v)
            pltpu.async_copy(t_hbm.at[idx_v], g_v, sem).wait()
            pltpu.sync_copy(g_v, out_hbm.at[pl.ds(off, chunk_len)])
            return carry

        lax.fori_loop(0, nch, chunk, 0)

    return k(t01, hseg)


def kernel(h, emb, W, b):
    B, L = h.shape
    t01 = _project_table(emb.T, W, b.reshape(1, -1))
    nc, ns = 2, 16
    nw = nc * ns
    total = B * L
    chunk_len = 2048
    nch = total // (nw * chunk_len)
    hseg = h.T.astype(jnp.int32).reshape(nw * nch * chunk_len)
    g = _sc_gather(t01, hseg, nc, ns, nch, chunk_len).reshape(L, B)
    f0 = jax.lax.bitcast_convert_type(g << 16, jnp.float32).T
    f1 = jax.lax.bitcast_convert_type(g & jnp.int32(-65536), jnp.float32).T
    return jnp.stack([f0, f1], axis=-1)


# speedup vs baseline: 4.4758x; 1.1169x over previous
import functools

import jax
import jax.numpy as jnp
from jax import lax
from jax.experimental import pallas as pl
from jax.experimental.pallas import tpu as pltpu
from jax.experimental.pallas import tpu_sc as plsc


def _project_table(embT, W, b2):
    hid, n = embT.shape
    blk = 32768

    def body(e_ref, w_ref, b_ref, o_ref):
        c = jax.lax.dot_general(
            w_ref[...], e_ref[...], (((1,), (0,)), ((), ())),
            preferred_element_type=jnp.float32,
        )
        r0 = jax.lax.bitcast_convert_type(c[0] + b_ref[0, 0], jnp.int32)
        r1 = jax.lax.bitcast_convert_type(c[1] + b_ref[0, 1], jnp.int32)
        h0 = (r0 + 0x7FFF + ((r0 >> 16) & 1)) >> 16
        h1 = (r1 + 0x7FFF + ((r1 >> 16) & 1)) >> 16
        o_ref[...] = (h1 << 16) | (h0 & 0xFFFF)

    return pl.pallas_call(
        body,
        grid=(pl.cdiv(n, blk),),
        in_specs=[
            pl.BlockSpec((hid, blk), lambda i: (0, i)),
            pl.BlockSpec((2, hid), lambda i: (0, 0)),
            pl.BlockSpec((1, 2), lambda i: (0, 0)),
        ],
        out_specs=pl.BlockSpec((blk,), lambda i: (i,)),
        out_shape=jax.ShapeDtypeStruct((n,), jnp.int32),
    )(embT, W, b2)


def _sc_gather(t01, hseg, nc, ns, nch, chunk_len):
    nw = nc * ns
    per_w = nch * chunk_len
    mesh = plsc.VectorSubcoreMesh(core_axis_name="c", subcore_axis_name="s")

    @functools.partial(
        pl.kernel,
        mesh=mesh,
        out_type=jax.ShapeDtypeStruct((nw * per_w,), jnp.int32),
        scratch_types=[
            pltpu.VMEM((chunk_len,), jnp.int32),
            pltpu.VMEM((chunk_len,), jnp.int32),
            pltpu.SemaphoreType.DMA,
        ],
        compiler_params=pltpu.CompilerParams(use_tc_tiling_on_sc=False),
    )
    def k(t_hbm, h_hbm, out_hbm, idx_v, g_v, sem):
        wid = lax.axis_index("s") * nc + lax.axis_index("c")
        base = wid * per_w

        def chunk(c, carry):
            off = base + c * chunk_len
            pltpu.sync_copy(h_hbm.at[pl.ds(off, chunk_len)], idx_v)
            pltpu.async_copy(t_hbm.at[idx_v], g_v, sem).wait()
            pltpu.sync_copy(g_v, out_hbm.at[pl.ds(off, chunk_len)])
            return carry

        lax.fori_loop(0, nch, chunk, 0)

    return k(t01, hseg)


def kernel(h, emb, W, b):
    B, L = h.shape
    t01 = _project_table(emb.T, W, b.reshape(1, -1))
    nc, ns = 2, 16
    nw = nc * ns
    total = B * L
    chunk_len = 2048
    nch = total // (nw * chunk_len)
    hseg = h.T.astype(jnp.int32).reshape(nw * nch * chunk_len)
    g = _sc_gather(t01, hseg, nc, ns, nch, chunk_len).reshape(L, B)
    f0 = jax.lax.bitcast_convert_type(g << 16, jnp.float32).T
    f1 = jax.lax.bitcast_convert_type(g & jnp.int32(-65536), jnp.float32).T
    return jnp.stack([f0, f1], axis=-1)


# speedup vs baseline: 4.4788x; 1.0007x over previous
import functools

import jax
import jax.numpy as jnp
from jax import lax
from jax.experimental import pallas as pl
from jax.experimental.pallas import tpu as pltpu
from jax.experimental.pallas import tpu_sc as plsc


def _project_table(embT, W, b2):
    hid, n = embT.shape
    blk = 65536

    def body(e_ref, w_ref, b_ref, o_ref):
        c = jax.lax.dot_general(
            w_ref[...], e_ref[...], (((1,), (0,)), ((), ())),
            preferred_element_type=jnp.float32,
        )
        r0 = jax.lax.bitcast_convert_type(c[0] + b_ref[0, 0], jnp.int32)
        r1 = jax.lax.bitcast_convert_type(c[1] + b_ref[0, 1], jnp.int32)
        h0 = (r0 + 0x7FFF + ((r0 >> 16) & 1)) >> 16
        h1 = (r1 + 0x7FFF + ((r1 >> 16) & 1)) >> 16
        o_ref[...] = (h1 << 16) | (h0 & 0xFFFF)

    return pl.pallas_call(
        body,
        grid=(pl.cdiv(n, blk),),
        in_specs=[
            pl.BlockSpec((hid, blk), lambda i: (0, i)),
            pl.BlockSpec((2, hid), lambda i: (0, 0)),
            pl.BlockSpec((1, 2), lambda i: (0, 0)),
        ],
        out_specs=pl.BlockSpec((blk,), lambda i: (i,)),
        out_shape=jax.ShapeDtypeStruct((n,), jnp.int32),
    )(embT, W, b2)


def _sc_gather(t01, hseg, nc, ns, nch, chunk_len):
    nw = nc * ns
    per_w = nch * chunk_len
    mesh = plsc.VectorSubcoreMesh(core_axis_name="c", subcore_axis_name="s")

    @functools.partial(
        pl.kernel,
        mesh=mesh,
        out_type=jax.ShapeDtypeStruct((nw * per_w,), jnp.int32),
        scratch_types=[
            pltpu.VMEM((chunk_len,), jnp.int32),
            pltpu.VMEM((chunk_len,), jnp.int32),
            pltpu.SemaphoreType.DMA,
        ],
        compiler_params=pltpu.CompilerParams(use_tc_tiling_on_sc=False),
    )
    def k(t_hbm, h_hbm, out_hbm, idx_v, g_v, sem):
        wid = lax.axis_index("s") * nc + lax.axis_index("c")
        base = wid * per_w

        def chunk(c, carry):
            off = base + c * chunk_len
            pltpu.sync_copy(h_hbm.at[pl.ds(off, chunk_len)], idx_v)
            pltpu.async_copy(t_hbm.at[idx_v], g_v, sem).wait()
            pltpu.sync_copy(g_v, out_hbm.at[pl.ds(off, chunk_len)])
            return carry

        lax.fori_loop(0, nch, chunk, 0)

    return k(t01, hseg)


def kernel(h, emb, W, b):
    B, L = h.shape
    t01 = _project_table(emb.T, W, b.reshape(1, -1))
    nc, ns = 2, 16
    nw = nc * ns
    total = B * L
    chunk_len = 2048
    nch = total // (nw * chunk_len)
    hseg = h.T.astype(jnp.int32).reshape(nw * nch * chunk_len)
    g = _sc_gather(t01, hseg, nc, ns, nch, chunk_len).reshape(L, B)
    f0 = jax.lax.bitcast_convert_type(g << 16, jnp.float32).T
    f1 = jax.lax.bitcast_convert_type(g & jnp.int32(-65536), jnp.float32).T
    return jnp.stack([f0, f1], axis=-1)


# speedup vs baseline: 5.3473x; 1.1939x over previous
import functools

import jax
import jax.numpy as jnp
from jax import lax
from jax.experimental import pallas as pl
from jax.experimental.pallas import tpu as pltpu
from jax.experimental.pallas import tpu_sc as plsc


def _project_table(embT, W, b2):
    hid, n = embT.shape
    blk = 65536

    def body(e_ref, w_ref, b_ref, o_ref):
        c = jax.lax.dot_general(
            w_ref[...], e_ref[...], (((1,), (0,)), ((), ())),
            preferred_element_type=jnp.float32,
        )
        r0 = jax.lax.bitcast_convert_type(c[0] + b_ref[0, 0], jnp.int32)
        r1 = jax.lax.bitcast_convert_type(c[1] + b_ref[0, 1], jnp.int32)
        h0 = (r0 + 0x7FFF + ((r0 >> 16) & 1)) >> 16
        h1 = (r1 + 0x7FFF + ((r1 >> 16) & 1)) >> 16
        o_ref[...] = (h1 << 16) | (h0 & 0xFFFF)

    return pl.pallas_call(
        body,
        grid=(pl.cdiv(n, blk),),
        in_specs=[
            pl.BlockSpec((hid, blk), lambda i: (0, i)),
            pl.BlockSpec((2, hid), lambda i: (0, 0)),
            pl.BlockSpec((1, 2), lambda i: (0, 0)),
        ],
        out_specs=pl.BlockSpec((blk,), lambda i: (i,)),
        out_shape=jax.ShapeDtypeStruct((n,), jnp.int32),
    )(embT, W, b2)


def _sc_gather(t01, hseg, nc, ns, nch, chunk_len):
    nw = nc * ns
    per_w = nch * chunk_len
    mesh = plsc.VectorSubcoreMesh(core_axis_name="c", subcore_axis_name="s")

    @functools.partial(
        pl.kernel,
        mesh=mesh,
        out_type=jax.ShapeDtypeStruct((nw * per_w,), jnp.int32),
        scratch_types=[
            pltpu.VMEM((chunk_len,), jnp.int32),
            pltpu.VMEM((chunk_len,), jnp.int32),
            pltpu.VMEM((chunk_len,), jnp.int32),
            pltpu.VMEM((chunk_len,), jnp.int32),
            pltpu.SemaphoreType.DMA,
            pltpu.SemaphoreType.DMA,
        ],
        compiler_params=pltpu.CompilerParams(use_tc_tiling_on_sc=False),
    )
    def k(t_hbm, h_hbm, out_hbm, idx_a, idx_b, g_a, g_b, sem_a, sem_b):
        wid = lax.axis_index("s") * nc + lax.axis_index("c")
        base = wid * per_w

        def load_idx(c, buf):
            pltpu.sync_copy(h_hbm.at[pl.ds(base + c * chunk_len, chunk_len)],
                            buf)

        def store_out(c, buf):
            pltpu.sync_copy(buf,
                            out_hbm.at[pl.ds(base + c * chunk_len, chunk_len)])

        load_idx(0, idx_a)

        def body(i, carry):
            c0 = 2 * i
            cp_a = pltpu.make_async_copy(t_hbm.at[idx_a], g_a, sem_a)
            cp_a.start()
            load_idx(c0 + 1, idx_b)
            cp_b = pltpu.make_async_copy(t_hbm.at[idx_b], g_b, sem_b)
            cp_b.start()
            cp_a.wait()
            store_out(c0, g_a)

            @pl.when(i + 1 < nch // 2)
            def _():
                load_idx(c0 + 2, idx_a)

            cp_b.wait()
            store_out(c0 + 1, g_b)
            return carry

        lax.fori_loop(0, nch // 2, body, 0)

    return k(t01, hseg)


def kernel(h, emb, W, b):
    B, L = h.shape
    t01 = _project_table(emb.T, W, b.reshape(1, -1))
    nc, ns = 2, 16
    nw = nc * ns
    total = B * L
    chunk_len = 2048
    nch = total // (nw * chunk_len)
    hseg = h.T.astype(jnp.int32).reshape(nw * nch * chunk_len)
    g = _sc_gather(t01, hseg, nc, ns, nch, chunk_len).reshape(L, B)
    f0 = jax.lax.bitcast_convert_type(g << 16, jnp.float32).T
    f1 = jax.lax.bitcast_convert_type(g & jnp.int32(-65536), jnp.float32).T
    return jnp.stack([f0, f1], axis=-1)


# speedup vs baseline: 5.3705x; 1.0044x over previous
import functools

import jax
import jax.numpy as jnp
from jax import lax
from jax.experimental import pallas as pl
from jax.experimental.pallas import tpu as pltpu
from jax.experimental.pallas import tpu_sc as plsc


def _project_table(embT, W, b2):
    hid, n = embT.shape
    blk = 65536

    def body(e_ref, w_ref, b_ref, o_ref):
        c = jax.lax.dot_general(
            w_ref[...], e_ref[...], (((1,), (0,)), ((), ())),
            preferred_element_type=jnp.float32,
        )
        r0 = jax.lax.bitcast_convert_type(c[0] + b_ref[0, 0], jnp.int32)
        r1 = jax.lax.bitcast_convert_type(c[1] + b_ref[0, 1], jnp.int32)
        h0 = (r0 + 0x7FFF + ((r0 >> 16) & 1)) >> 16
        h1 = (r1 + 0x7FFF + ((r1 >> 16) & 1)) >> 16
        o_ref[...] = (h1 << 16) | (h0 & 0xFFFF)

    return pl.pallas_call(
        body,
        grid=(pl.cdiv(n, blk),),
        in_specs=[
            pl.BlockSpec((hid, blk), lambda i: (0, i)),
            pl.BlockSpec((2, hid), lambda i: (0, 0)),
            pl.BlockSpec((1, 2), lambda i: (0, 0)),
        ],
        out_specs=pl.BlockSpec((blk,), lambda i: (i,)),
        out_shape=jax.ShapeDtypeStruct((n,), jnp.int32),
    )(embT, W, b2)


def _sc_gather(t01, hseg, nc, ns, nch, chunk_len):
    nw = nc * ns
    per_w = nch * chunk_len
    mesh = plsc.VectorSubcoreMesh(core_axis_name="c", subcore_axis_name="s")

    @functools.partial(
        pl.kernel,
        mesh=mesh,
        out_type=jax.ShapeDtypeStruct((nw * per_w,), jnp.int32),
        scratch_types=[
            pltpu.VMEM((chunk_len,), jnp.int32),
            pltpu.VMEM((chunk_len,), jnp.int32),
            pltpu.VMEM((chunk_len,), jnp.int32),
            pltpu.VMEM((chunk_len,), jnp.int32),
            pltpu.SemaphoreType.DMA,
            pltpu.SemaphoreType.DMA,
        ],
        compiler_params=pltpu.CompilerParams(use_tc_tiling_on_sc=False),
    )
    def k(t_hbm, h_hbm, out_hbm, idx_a, idx_b, g_a, g_b, sem_a, sem_b):
        wid = lax.axis_index("s") * nc + lax.axis_index("c")
        base = wid * per_w

        def load_idx(c, buf):
            pltpu.sync_copy(h_hbm.at[pl.ds(base + c * chunk_len, chunk_len)],
                            buf)

        def store_out(c, buf):
            pltpu.sync_copy(buf,
                            out_hbm.at[pl.ds(base + c * chunk_len, chunk_len)])

        load_idx(0, idx_a)

        def body(i, carry):
            c0 = 2 * i
            cp_a = pltpu.make_async_copy(t_hbm.at[idx_a], g_a, sem_a)
            cp_a.start()
            load_idx(c0 + 1, idx_b)
            cp_b = pltpu.make_async_copy(t_hbm.at[idx_b], g_b, sem_b)
            cp_b.start()
            cp_a.wait()
            store_out(c0, g_a)

            @pl.when(i + 1 < nch // 2)
            def _():
                load_idx(c0 + 2, idx_a)

            cp_b.wait()
            store_out(c0 + 1, g_b)
            return carry

        lax.fori_loop(0, nch // 2, body, 0)

    return k(t01, hseg)


def kernel(h, emb, W, b):
    B, L = h.shape
    t01 = _project_table(emb.T, W, b.reshape(1, -1))
    nc, ns = 2, 16
    nw = nc * ns
    total = B * L
    chunk_len = 6400
    nch = total // (nw * chunk_len)
    hseg = h.T.astype(jnp.int32).reshape(nw * nch * chunk_len)
    g = _sc_gather(t01, hseg, nc, ns, nch, chunk_len).reshape(L, B)
    f0 = jax.lax.bitcast_convert_type(g << 16, jnp.float32).T
    f1 = jax.lax.bitcast_convert_type(g & jnp.int32(-65536), jnp.float32).T
    return jnp.stack([f0, f1], axis=-1)


# speedup vs baseline: 6.1249x; 1.1405x over previous
import functools

import jax
import jax.numpy as jnp
from jax import lax
from jax.experimental import pallas as pl
from jax.experimental.pallas import tpu as pltpu
from jax.experimental.pallas import tpu_sc as plsc


def _project_table(embT, W, b2):
    hid, n = embT.shape
    blk = 65536

    def body(e_ref, w_ref, b_ref, o_ref):
        c = jax.lax.dot_general(
            w_ref[...], e_ref[...], (((1,), (0,)), ((), ())),
            preferred_element_type=jnp.float32,
        )
        r0 = jax.lax.bitcast_convert_type(c[0] + b_ref[0, 0], jnp.int32)
        r1 = jax.lax.bitcast_convert_type(c[1] + b_ref[0, 1], jnp.int32)
        h0 = (r0 + 0x7FFF + ((r0 >> 16) & 1)) >> 16
        h1 = (r1 + 0x7FFF + ((r1 >> 16) & 1)) >> 16
        o_ref[...] = (h1 << 16) | (h0 & 0xFFFF)

    return pl.pallas_call(
        body,
        grid=(pl.cdiv(n, blk),),
        in_specs=[
            pl.BlockSpec((hid, blk), lambda i: (0, i)),
            pl.BlockSpec((2, hid), lambda i: (0, 0)),
            pl.BlockSpec((1, 2), lambda i: (0, 0)),
        ],
        out_specs=pl.BlockSpec((blk,), lambda i: (i,)),
        out_shape=jax.ShapeDtypeStruct((n,), jnp.int32),
    )(embT, W, b2)


def _sc_gather(t01, hseg, nc, ns, nch, chunk_len):
    nw = nc * ns
    per_w = nch * chunk_len
    mesh = plsc.VectorSubcoreMesh(core_axis_name="c", subcore_axis_name="s")

    @functools.partial(
        pl.kernel,
        mesh=mesh,
        out_type=jax.ShapeDtypeStruct((nw * per_w,), jnp.int32),
        scratch_types=[
            pltpu.VMEM((chunk_len,), jnp.int32),
            pltpu.VMEM((chunk_len,), jnp.int32),
            pltpu.VMEM((chunk_len,), jnp.int32),
            pltpu.VMEM((chunk_len,), jnp.int32),
            pltpu.SemaphoreType.DMA,
            pltpu.SemaphoreType.DMA,
        ],
        compiler_params=pltpu.CompilerParams(use_tc_tiling_on_sc=False),
    )
    def k(t_hbm, h_hbm, out_hbm, idx_a, idx_b, g_a, g_b, sem_a, sem_b):
        wid = lax.axis_index("s") * nc + lax.axis_index("c")
        base = wid * per_w

        def load_idx(c, buf):
            pltpu.sync_copy(h_hbm.at[pl.ds(base + c * chunk_len, chunk_len)],
                            buf)

        def store_out(c, buf):
            pltpu.sync_copy(buf,
                            out_hbm.at[pl.ds(base + c * chunk_len, chunk_len)])

        load_idx(0, idx_a)

        def body(i, carry):
            c0 = 2 * i
            cp_a = pltpu.make_async_copy(t_hbm.at[idx_a], g_a, sem_a)
            cp_a.start()
            load_idx(c0 + 1, idx_b)
            cp_b = pltpu.make_async_copy(t_hbm.at[idx_b], g_b, sem_b)
            cp_b.start()
            cp_a.wait()
            store_out(c0, g_a)

            @pl.when(i + 1 < nch // 2)
            def _():
                load_idx(c0 + 2, idx_a)

            cp_b.wait()
            store_out(c0 + 1, g_b)
            return carry

        lax.fori_loop(0, nch // 2, body, 0)

    return k(t01, hseg)


def kernel(h, emb, W, b):
    B, L = h.shape
    t01 = _project_table(emb.T, W, b.reshape(1, -1))
    nc, ns = 2, 16
    nw = nc * ns
    total = B * L
    chunk_len = 6400
    nch = total // (nw * chunk_len)
    hseg = h.T.astype(jnp.int32).reshape(nw * nch * chunk_len)
    g = _sc_gather(t01, hseg, nc, ns, nch, chunk_len).reshape(L, B)
    halves = jax.lax.bitcast_convert_type(g, jnp.int16)
    f = jax.lax.bitcast_convert_type(
        halves.astype(jnp.int32) << 16, jnp.float32)
    return jnp.transpose(f, (1, 0, 2))
